# Initial kernel scaffold; baseline (speedup 1.0000x reference)
import functools
import math

import jax
import jax.numpy as jnp
from jax import lax
from jax.experimental import pallas as pl
from jax.experimental.pallas import tpu as pltpu
from jax.experimental.pallas import tpu_sc as plsc

N = 10000
E = 160000
IN = 32
OUT = 32
NEF = 16
TD = 128
INV_SQRT_FAN = 1.0 / math.sqrt(IN * 1)

NC = 2
NS = 16
NW = NC * NS
CHUNK = 128
NCHUNKS = E // CHUNK
ROWS_PER_TILE = N // NS

BE = 2000


def _node_gates_body(t_ref, na_ref, wtin_ref, btin_ref, wtout_ref, btout_ref,
                     nag_ref, go_ref):
    t = t_ref[...]
    st = t * jax.nn.sigmoid(t)
    gi = jnp.dot(st, wtin_ref[...], preferred_element_type=jnp.float32) + btin_ref[...]
    go = jnp.dot(st, wtout_ref[...], preferred_element_type=jnp.float32) + btout_ref[...]
    nag_ref[...] = na_ref[...] * gi
    go_ref[...] = go


def _node_gates(time, node_attr, W_tin, b_tin, W_tout, b_tout):
    return pl.pallas_call(
        _node_gates_body,
        out_shape=(
            jax.ShapeDtypeStruct((N, IN), jnp.float32),
            jax.ShapeDtypeStruct((N, OUT), jnp.float32),
        ),
    )(time, node_attr, W_tin, b_tin.reshape(1, IN), W_tout, b_tout.reshape(1, OUT))


def _gather_body(na_hbm, idx_hbm, out_hbm, idx_v, rows_v, sem):
    wid = lax.axis_index("s") * NC + lax.axis_index("c")
    nch = NCHUNKS // NW + jnp.where(wid < NCHUNKS % NW, 1, 0)

    def body(c, carry):
        base = (wid + NW * c) * CHUNK
        pltpu.sync_copy(idx_hbm.at[pl.ds(base, CHUNK)], idx_v)
        pltpu.async_copy(na_hbm.at[idx_v], rows_v, sem).wait()
        pltpu.sync_copy(rows_v, out_hbm.at[pl.ds(base, CHUNK)])
        return carry

    lax.fori_loop(0, nch, body, 0)


def _gather_rows(na, dst):
    mesh = plsc.VectorSubcoreMesh(core_axis_name="c", subcore_axis_name="s")
    f = functools.partial(
        pl.kernel,
        out_type=jax.ShapeDtypeStruct((E, IN), jnp.float32),
        mesh=mesh,
        scratch_types=[
            pltpu.VMEM((CHUNK,), jnp.int32),
            pltpu.VMEM((CHUNK, IN), jnp.float32),
            pltpu.SemaphoreType.DMA,
        ],
    )(_gather_body)
    return f(na, dst)


def _edge_body(ea_ref, et_ref, sh_ref, xg_ref, wcat_ref, bcat_ref, w1_ref,
               b1_ref, w2_ref, tp_ref):
    et = et_ref[...]
    st = et * jax.nn.sigmoid(et)
    cat = jnp.concatenate([ea_ref[...], st], axis=1)
    pre = jnp.dot(cat, wcat_ref[...], preferred_element_type=jnp.float32)
    pre = pre + bcat_ref[...]
    ea = pre[:, :NEF]
    shift = pre[:, NEF:2 * NEF]
    scale = pre[:, 2 * NEF:3 * NEF]
    ea = ea * (1.0 + scale) + shift
    h = jnp.dot(ea, w1_ref[...], preferred_element_type=jnp.float32) + b1_ref[...]
    h = jnp.maximum(h, 0.0)
    x = xg_ref[...] * (sh_ref[...] * INV_SQRT_FAN)
    g = jnp.dot(x, w2_ref[...], preferred_element_type=jnp.float32)
    acc = g[:, NEF * OUT:]
    for j in range(NEF):
        acc = acc + g[:, j * OUT:(j + 1) * OUT] * h[:, j:j + 1]
    tp_ref[...] = acc


def _edge_tp(edge_attr, edge_time, edge_sh, xg, Wcat, bcat, W_fc1, b_fc1, W2aug):
    grid = (E // BE,)
    return pl.pallas_call(
        _edge_body,
        grid=grid,
        in_specs=[
            pl.BlockSpec((BE, NEF), lambda i: (i, 0)),
            pl.BlockSpec((BE, TD), lambda i: (i, 0)),
            pl.BlockSpec((BE, 1), lambda i: (i, 0)),
            pl.BlockSpec((BE, IN), lambda i: (i, 0)),
            pl.BlockSpec((NEF + TD, 3 * NEF), lambda i: (0, 0)),
            pl.BlockSpec((1, 3 * NEF), lambda i: (0, 0)),
            pl.BlockSpec((NEF, NEF), lambda i: (0, 0)),
            pl.BlockSpec((1, NEF), lambda i: (0, 0)),
            pl.BlockSpec((IN, (NEF + 1) * OUT), lambda i: (0, 0)),
        ],
        out_specs=pl.BlockSpec((BE, OUT), lambda i: (i, 0)),
        out_shape=jax.ShapeDtypeStruct((E, OUT), jnp.float32),
    )(edge_attr, edge_time, edge_sh, xg, Wcat, bcat, W_fc1, b_fc1, W2aug)


def _scatter_body(tp_hbm, src_hbm, zsum_hbm, zcnt_hbm, sums_hbm, cnts_hbm,
                  idx_v, tp_v, ones_v, acc_sh, cnt_sh):
    cid = lax.axis_index("c")
    sid = lax.axis_index("s")
    wid = sid * NC + cid
    row0 = sid * ROWS_PER_TILE

    def fill(i, carry):
        ones_v[i, :] = jnp.full((16,), 1.0, jnp.float32)
        return carry
    lax.fori_loop(0, CHUNK, fill, 0)

    pltpu.sync_copy(zsum_hbm.at[pl.ds(row0, ROWS_PER_TILE)],
                    acc_sh.at[pl.ds(row0, ROWS_PER_TILE)])
    pltpu.sync_copy(zcnt_hbm.at[pl.ds(row0, ROWS_PER_TILE)],
                    cnt_sh.at[pl.ds(row0, ROWS_PER_TILE)])
    plsc.subcore_barrier()

    nch = NCHUNKS // NW + jnp.where(wid < NCHUNKS % NW, 1, 0)

    def body(c, carry):
        base = (wid + NW * c) * CHUNK
        pltpu.sync_copy(src_hbm.at[pl.ds(base, CHUNK)], idx_v)
        pltpu.sync_copy(tp_hbm.at[pl.ds(base, CHUNK)], tp_v)
        pltpu.sync_copy(tp_v, acc_sh.at[idx_v], add=True)
        pltpu.sync_copy(ones_v, cnt_sh.at[idx_v], add=True)
        return carry

    lax.fori_loop(0, nch, body, 0)
    plsc.subcore_barrier()

    pltpu.sync_copy(acc_sh.at[pl.ds(row0, ROWS_PER_TILE)],
                    sums_hbm.at[cid, pl.ds(row0, ROWS_PER_TILE)])
    pltpu.sync_copy(cnt_sh.at[pl.ds(row0, ROWS_PER_TILE)],
                    cnts_hbm.at[cid, pl.ds(row0, ROWS_PER_TILE)])


def _scatter_mean_partials(tp, src):
    mesh = plsc.VectorSubcoreMesh(core_axis_name="c", subcore_axis_name="s")
    zsum = jnp.zeros((N, OUT), jnp.float32)
    zcnt = jnp.zeros((N, 16), jnp.float32)
    f = functools.partial(
        pl.kernel,
        out_type=(
            jax.ShapeDtypeStruct((NC, N, OUT), jnp.float32),
            jax.ShapeDtypeStruct((NC, N, 16), jnp.float32),
        ),
        mesh=mesh,
        scratch_types=[
            pltpu.VMEM((CHUNK,), jnp.int32),
            pltpu.VMEM((CHUNK, OUT), jnp.float32),
            pltpu.VMEM((CHUNK, 16), jnp.float32),
            pltpu.VMEM_SHARED((N, OUT), jnp.float32),
            pltpu.VMEM_SHARED((N, 16), jnp.float32),
        ],
    )(_scatter_body)
    return f(tp, src, zsum, zcnt)


def _final_body(s_ref, c_ref, go_ref, na_ref, out_ref):
    s = s_ref[0] + s_ref[1]
    c = c_ref[0] + c_ref[1]
    cnt = jnp.clip(c[:, 0:1], 1.0, None)
    out_ref[...] = s / cnt * go_ref[...] + na_ref[...]


def _finalize(sums, cnts, go, node_attr):
    return pl.pallas_call(
        _final_body,
        out_shape=jax.ShapeDtypeStruct((N, OUT), jnp.float32),
    )(sums, cnts, go, node_attr)


def kernel(node_attr, edge_index, edge_attr, edge_sh, time, edge_time,
           W_fc_pre, b_fc_pre, W_ada, b_ada, W_fc1, b_fc1, W_fc2, b_fc2,
           W_tin, b_tin, W_tout, b_tout):
    Wcat = jnp.zeros((NEF + TD, 3 * NEF), jnp.float32)
    Wcat = Wcat.at[:NEF, :NEF].set(W_fc_pre).at[NEF:, NEF:].set(W_ada)
    bcat = jnp.concatenate([b_fc_pre, b_ada]).reshape(1, 3 * NEF)
    W2p = W_fc2.reshape(NEF, IN, OUT).transpose(1, 0, 2).reshape(IN, NEF * OUT)
    W2aug = jnp.concatenate([W2p, b_fc2.reshape(IN, OUT)], axis=1)

    src = edge_index[0]
    dst = edge_index[1]

    na, go = _node_gates(time, node_attr, W_tin, b_tin, W_tout, b_tout)
    xg = _gather_rows(na, dst)
    tp = _edge_tp(edge_attr, edge_time, edge_sh, xg, Wcat, bcat, W_fc1, b_fc1,
                  W2aug)
    sums, cnts = _scatter_mean_partials(tp, src)
    return _finalize(sums, cnts, go, node_attr)


# speedup vs baseline: 2.0945x; 2.0945x over previous
import functools
import math

import jax
import jax.numpy as jnp
from jax import lax
from jax.experimental import pallas as pl
from jax.experimental.pallas import tpu as pltpu
from jax.experimental.pallas import tpu_sc as plsc

N = 10000
E = 160000
IN = 32
OUT = 32
NEF = 16
TD = 128
INV_SQRT_FAN = 1.0 / math.sqrt(IN * 1)

NC = 2
NS = 16
NW = NC * NS
CHUNK = 128
NCHUNKS = E // CHUNK
ROWS_PER_TILE = N // NS

BE = 2000


def _node_gates_body(t_ref, na_ref, wtin_ref, btin_ref, wtout_ref, btout_ref,
                     nag_ref, go_ref):
    t = t_ref[...]
    st = t * jax.nn.sigmoid(t)
    gi = jnp.dot(st, wtin_ref[...], preferred_element_type=jnp.float32) + btin_ref[...]
    go = jnp.dot(st, wtout_ref[...], preferred_element_type=jnp.float32) + btout_ref[...]
    nag_ref[...] = na_ref[...] * gi
    go_ref[...] = go


def _node_gates(time, node_attr, W_tin, b_tin, W_tout, b_tout):
    return pl.pallas_call(
        _node_gates_body,
        out_shape=(
            jax.ShapeDtypeStruct((N, IN), jnp.float32),
            jax.ShapeDtypeStruct((N, OUT), jnp.float32),
        ),
    )(time, node_attr, W_tin, b_tin.reshape(1, IN), W_tout, b_tout.reshape(1, OUT))


def _gather_body(na_hbm, idx_hbm, out_hbm, idx_v, rows_v, sem):
    wid = lax.axis_index("s") * NC + lax.axis_index("c")
    nch = NCHUNKS // NW + jnp.where(wid < NCHUNKS % NW, 1, 0)

    def body(c, carry):
        base = (wid + NW * c) * CHUNK
        pltpu.sync_copy(idx_hbm.at[pl.ds(base, CHUNK)], idx_v)
        pltpu.async_copy(na_hbm.at[idx_v], rows_v, sem).wait()
        pltpu.sync_copy(rows_v, out_hbm.at[pl.ds(base, CHUNK)])
        return carry

    lax.fori_loop(0, nch, body, 0)


def _gather_rows(na, dst):
    mesh = plsc.VectorSubcoreMesh(core_axis_name="c", subcore_axis_name="s")
    f = functools.partial(
        pl.kernel,
        out_type=jax.ShapeDtypeStruct((E, IN), jnp.float32),
        mesh=mesh,
        scratch_types=[
            pltpu.VMEM((CHUNK,), jnp.int32),
            pltpu.VMEM((CHUNK, IN), jnp.float32),
            pltpu.SemaphoreType.DMA,
        ],
        compiler_params=pltpu.CompilerParams(use_tc_tiling_on_sc=False),
    )(_gather_body)
    return f(na, dst)


def _edge_body(ea_ref, et_ref, sh_ref, xg_ref, wcat_ref, bcat_ref, w1_ref,
               b1_ref, w2_ref, tp_ref):
    et = et_ref[...]
    st = et * jax.nn.sigmoid(et)
    cat = jnp.concatenate([ea_ref[...], st], axis=1)
    pre = jnp.dot(cat, wcat_ref[...], preferred_element_type=jnp.float32)
    pre = pre + bcat_ref[...]
    ea = pre[:, :NEF]
    shift = pre[:, NEF:2 * NEF]
    scale = pre[:, 2 * NEF:3 * NEF]
    ea = ea * (1.0 + scale) + shift
    h = jnp.dot(ea, w1_ref[...], preferred_element_type=jnp.float32) + b1_ref[...]
    h = jnp.maximum(h, 0.0)
    x = xg_ref[...] * (sh_ref[...] * INV_SQRT_FAN)
    g = jnp.dot(x, w2_ref[...], preferred_element_type=jnp.float32)
    acc = g[:, NEF * OUT:]
    for j in range(NEF):
        acc = acc + g[:, j * OUT:(j + 1) * OUT] * h[:, j:j + 1]
    tp_ref[...] = acc


def _edge_tp(edge_attr, edge_time, edge_sh, xg, Wcat, bcat, W_fc1, b_fc1, W2aug):
    grid = (E // BE,)
    return pl.pallas_call(
        _edge_body,
        grid=grid,
        in_specs=[
            pl.BlockSpec((BE, NEF), lambda i: (i, 0)),
            pl.BlockSpec((BE, TD), lambda i: (i, 0)),
            pl.BlockSpec((BE, 1), lambda i: (i, 0)),
            pl.BlockSpec((BE, IN), lambda i: (i, 0)),
            pl.BlockSpec((NEF + TD, 3 * NEF), lambda i: (0, 0)),
            pl.BlockSpec((1, 3 * NEF), lambda i: (0, 0)),
            pl.BlockSpec((NEF, NEF), lambda i: (0, 0)),
            pl.BlockSpec((1, NEF), lambda i: (0, 0)),
            pl.BlockSpec((IN, (NEF + 1) * OUT), lambda i: (0, 0)),
        ],
        out_specs=pl.BlockSpec((BE, OUT), lambda i: (i, 0)),
        out_shape=jax.ShapeDtypeStruct((E, OUT), jnp.float32),
    )(edge_attr, edge_time, edge_sh, xg, Wcat, bcat, W_fc1,
      b_fc1.reshape(1, NEF), W2aug)


def _scatter_body(tp_hbm, src_hbm, zsum_hbm, zcnt_hbm, sums_hbm, cnts_hbm,
                  idx_v, tp_v, ones_v, acc_sh, cnt_sh):
    cid = lax.axis_index("c")
    sid = lax.axis_index("s")
    wid = sid * NC + cid
    row0 = sid * ROWS_PER_TILE

    def fill(i, carry):
        ones_v[i, :] = jnp.full((16,), 1.0, jnp.float32)
        return carry
    lax.fori_loop(0, CHUNK, fill, 0)

    pltpu.sync_copy(zsum_hbm.at[pl.ds(row0, ROWS_PER_TILE)],
                    acc_sh.at[pl.ds(row0, ROWS_PER_TILE)])
    pltpu.sync_copy(zcnt_hbm.at[pl.ds(row0, ROWS_PER_TILE)],
                    cnt_sh.at[pl.ds(row0, ROWS_PER_TILE)])
    plsc.subcore_barrier()

    nch = NCHUNKS // NW + jnp.where(wid < NCHUNKS % NW, 1, 0)

    def body(c, carry):
        base = (wid + NW * c) * CHUNK
        pltpu.sync_copy(src_hbm.at[pl.ds(base, CHUNK)], idx_v)
        pltpu.sync_copy(tp_hbm.at[pl.ds(base, CHUNK)], tp_v)
        pltpu.sync_copy(tp_v, acc_sh.at[idx_v], add=True)
        pltpu.sync_copy(ones_v, cnt_sh.at[idx_v], add=True)
        return carry

    lax.fori_loop(0, nch, body, 0)
    plsc.subcore_barrier()

    pltpu.sync_copy(acc_sh.at[pl.ds(row0, ROWS_PER_TILE)],
                    sums_hbm.at[cid, pl.ds(row0, ROWS_PER_TILE)])
    pltpu.sync_copy(cnt_sh.at[pl.ds(row0, ROWS_PER_TILE)],
                    cnts_hbm.at[cid, pl.ds(row0, ROWS_PER_TILE)])


def _scatter_mean_partials(tp, src):
    mesh = plsc.VectorSubcoreMesh(core_axis_name="c", subcore_axis_name="s")
    zsum = jnp.zeros((N, OUT), jnp.float32)
    zcnt = jnp.zeros((N, 16), jnp.float32)
    f = functools.partial(
        pl.kernel,
        out_type=(
            jax.ShapeDtypeStruct((NC, N, OUT), jnp.float32),
            jax.ShapeDtypeStruct((NC, N, 16), jnp.float32),
        ),
        mesh=mesh,
        scratch_types=[
            pltpu.VMEM((CHUNK,), jnp.int32),
            pltpu.VMEM((CHUNK, OUT), jnp.float32),
            pltpu.VMEM((CHUNK, 16), jnp.float32),
            pltpu.VMEM_SHARED((N, OUT), jnp.float32),
            pltpu.VMEM_SHARED((N, 16), jnp.float32),
        ],
        compiler_params=pltpu.CompilerParams(use_tc_tiling_on_sc=False),
    )(_scatter_body)
    return f(tp, src, zsum, zcnt)


def _final_body(s_ref, c_ref, go_ref, na_ref, out_ref):
    s = s_ref[0] + s_ref[1]
    c = c_ref[0] + c_ref[1]
    cnt = jnp.clip(c[:, 0:1], 1.0, None)
    out_ref[...] = s / cnt * go_ref[...] + na_ref[...]


def _finalize(sums, cnts, go, node_attr):
    return pl.pallas_call(
        _final_body,
        out_shape=jax.ShapeDtypeStruct((N, OUT), jnp.float32),
    )(sums, cnts, go, node_attr)


def kernel(node_attr, edge_index, edge_attr, edge_sh, time, edge_time,
           W_fc_pre, b_fc_pre, W_ada, b_ada, W_fc1, b_fc1, W_fc2, b_fc2,
           W_tin, b_tin, W_tout, b_tout):
    Wcat = jnp.zeros((NEF + TD, 3 * NEF), jnp.float32)
    Wcat = Wcat.at[:NEF, :NEF].set(W_fc_pre).at[NEF:, NEF:].set(W_ada)
    bcat = jnp.concatenate([b_fc_pre, b_ada]).reshape(1, 3 * NEF)
    W2p = W_fc2.reshape(NEF, IN, OUT).transpose(1, 0, 2).reshape(IN, NEF * OUT)
    W2aug = jnp.concatenate([W2p, b_fc2.reshape(IN, OUT)], axis=1)

    src = edge_index[0]
    dst = edge_index[1]

    na, go = _node_gates(time, node_attr, W_tin, b_tin, W_tout, b_tout)
    xg = _gather_rows(na, dst)
    tp = _edge_tp(edge_attr, edge_time, edge_sh, xg, Wcat, bcat, W_fc1, b_fc1,
                  W2aug)
    sums, cnts = _scatter_mean_partials(tp, src)
    return _finalize(sums, cnts, go, node_attr)


# speedup vs baseline: 2.1029x; 1.0040x over previous
import functools
import math

import jax
import jax.numpy as jnp
from jax import lax
from jax.experimental import pallas as pl
from jax.experimental.pallas import tpu as pltpu
from jax.experimental.pallas import tpu_sc as plsc

N = 10000
E = 160000
IN = 32
OUT = 32
NEF = 16
TD = 128
INV_SQRT_FAN = 1.0 / math.sqrt(IN * 1)

NC = 2
NS = 16
NW = NC * NS
CHUNK = 128
NCHUNKS = E // CHUNK
ROWS_PER_TILE = N // NS

BE = 4000


def _node_gates_body(t_ref, na_ref, wtin_ref, btin_ref, wtout_ref, btout_ref,
                     nag_ref, go_ref):
    t = t_ref[...]
    st = t * jax.nn.sigmoid(t)
    gi = jnp.dot(st, wtin_ref[...], preferred_element_type=jnp.float32) + btin_ref[...]
    go = jnp.dot(st, wtout_ref[...], preferred_element_type=jnp.float32) + btout_ref[...]
    nag_ref[...] = na_ref[...] * gi
    go_ref[...] = go


def _node_gates(time, node_attr, W_tin, b_tin, W_tout, b_tout):
    return pl.pallas_call(
        _node_gates_body,
        out_shape=(
            jax.ShapeDtypeStruct((N, IN), jnp.float32),
            jax.ShapeDtypeStruct((N, OUT), jnp.float32),
        ),
    )(time, node_attr, W_tin, b_tin.reshape(1, IN), W_tout, b_tout.reshape(1, OUT))


def _gather_body(na_hbm, idx_hbm, out_hbm, idx_v, rows_v, sem):
    wid = lax.axis_index("s") * NC + lax.axis_index("c")
    nch = NCHUNKS // NW + jnp.where(wid < NCHUNKS % NW, 1, 0)

    def body(c, carry):
        base = (wid + NW * c) * CHUNK
        pltpu.sync_copy(idx_hbm.at[pl.ds(base, CHUNK)], idx_v)
        pltpu.async_copy(na_hbm.at[idx_v], rows_v, sem).wait()
        pltpu.sync_copy(rows_v, out_hbm.at[pl.ds(base, CHUNK)])
        return carry

    lax.fori_loop(0, nch, body, 0)


def _gather_rows(na, dst):
    mesh = plsc.VectorSubcoreMesh(core_axis_name="c", subcore_axis_name="s")
    f = functools.partial(
        pl.kernel,
        out_type=jax.ShapeDtypeStruct((E, IN), jnp.float32),
        mesh=mesh,
        scratch_types=[
            pltpu.VMEM((CHUNK,), jnp.int32),
            pltpu.VMEM((CHUNK, IN), jnp.float32),
            pltpu.SemaphoreType.DMA,
        ],
        compiler_params=pltpu.CompilerParams(use_tc_tiling_on_sc=False),
    )(_gather_body)
    return f(na, dst)


def _edge_body(ea_ref, et_ref, sh_ref, xg_ref, wcat_ref, bcat_ref, w1_ref,
               b1_ref, w2_ref, tp_ref):
    et = et_ref[...]
    st = et * jax.nn.sigmoid(et)
    cat = jnp.concatenate([ea_ref[...], st], axis=1)
    pre = jnp.dot(cat, wcat_ref[...], preferred_element_type=jnp.float32)
    pre = pre + bcat_ref[...]
    ea = pre[:, :NEF]
    shift = pre[:, NEF:2 * NEF]
    scale = pre[:, 2 * NEF:3 * NEF]
    ea = ea * (1.0 + scale) + shift
    h = jnp.dot(ea, w1_ref[...], preferred_element_type=jnp.float32) + b1_ref[...]
    h = jnp.maximum(h, 0.0)
    x = xg_ref[...] * (sh_ref[...] * INV_SQRT_FAN)
    g = jnp.dot(x, w2_ref[...], preferred_element_type=jnp.float32)
    acc = g[:, NEF * OUT:]
    for j in range(NEF):
        acc = acc + g[:, j * OUT:(j + 1) * OUT] * h[:, j:j + 1]
    tp_ref[...] = acc


def _edge_tp(edge_attr, edge_time, edge_sh, xg, Wcat, bcat, W_fc1, b_fc1, W2aug):
    grid = (E // BE,)
    return pl.pallas_call(
        _edge_body,
        grid=grid,
        in_specs=[
            pl.BlockSpec((BE, NEF), lambda i: (i, 0)),
            pl.BlockSpec((BE, TD), lambda i: (i, 0)),
            pl.BlockSpec((BE, 1), lambda i: (i, 0)),
            pl.BlockSpec((BE, IN), lambda i: (i, 0)),
            pl.BlockSpec((NEF + TD, 3 * NEF), lambda i: (0, 0)),
            pl.BlockSpec((1, 3 * NEF), lambda i: (0, 0)),
            pl.BlockSpec((NEF, NEF), lambda i: (0, 0)),
            pl.BlockSpec((1, NEF), lambda i: (0, 0)),
            pl.BlockSpec((IN, (NEF + 1) * OUT), lambda i: (0, 0)),
        ],
        out_specs=pl.BlockSpec((BE, OUT), lambda i: (i, 0)),
        out_shape=jax.ShapeDtypeStruct((E, OUT), jnp.float32),
    )(edge_attr, edge_time, edge_sh, xg, Wcat, bcat, W_fc1,
      b_fc1.reshape(1, NEF), W2aug)


def _scatter_body(tp_hbm, src_hbm, zsum_hbm, zcnt_hbm, sums_hbm, cnts_hbm,
                  idx_v, tp_v, ones_v, acc_sh, cnt_sh):
    cid = lax.axis_index("c")
    sid = lax.axis_index("s")
    wid = sid * NC + cid
    row0 = sid * ROWS_PER_TILE

    def fill(i, carry):
        ones_v[i, :] = jnp.full((16,), 1.0, jnp.float32)
        return carry
    lax.fori_loop(0, CHUNK, fill, 0)

    pltpu.sync_copy(zsum_hbm.at[pl.ds(row0, ROWS_PER_TILE)],
                    acc_sh.at[pl.ds(row0, ROWS_PER_TILE)])
    pltpu.sync_copy(zcnt_hbm.at[pl.ds(row0, ROWS_PER_TILE)],
                    cnt_sh.at[pl.ds(row0, ROWS_PER_TILE)])
    plsc.subcore_barrier()

    nch = NCHUNKS // NW + jnp.where(wid < NCHUNKS % NW, 1, 0)

    def body(c, carry):
        base = (wid + NW * c) * CHUNK
        pltpu.sync_copy(src_hbm.at[pl.ds(base, CHUNK)], idx_v)
        pltpu.sync_copy(tp_hbm.at[pl.ds(base, CHUNK)], tp_v)
        pltpu.sync_copy(tp_v, acc_sh.at[idx_v], add=True)
        pltpu.sync_copy(ones_v, cnt_sh.at[idx_v], add=True)
        return carry

    lax.fori_loop(0, nch, body, 0)
    plsc.subcore_barrier()

    pltpu.sync_copy(acc_sh.at[pl.ds(row0, ROWS_PER_TILE)],
                    sums_hbm.at[cid, pl.ds(row0, ROWS_PER_TILE)])
    pltpu.sync_copy(cnt_sh.at[pl.ds(row0, ROWS_PER_TILE)],
                    cnts_hbm.at[cid, pl.ds(row0, ROWS_PER_TILE)])


def _scatter_mean_partials(tp, src):
    mesh = plsc.VectorSubcoreMesh(core_axis_name="c", subcore_axis_name="s")
    zsum = jnp.zeros((N, OUT), jnp.float32)
    zcnt = jnp.zeros((N, 16), jnp.float32)
    f = functools.partial(
        pl.kernel,
        out_type=(
            jax.ShapeDtypeStruct((NC, N, OUT), jnp.float32),
            jax.ShapeDtypeStruct((NC, N, 16), jnp.float32),
        ),
        mesh=mesh,
        scratch_types=[
            pltpu.VMEM((CHUNK,), jnp.int32),
            pltpu.VMEM((CHUNK, OUT), jnp.float32),
            pltpu.VMEM((CHUNK, 16), jnp.float32),
            pltpu.VMEM_SHARED((N, OUT), jnp.float32),
            pltpu.VMEM_SHARED((N, 16), jnp.float32),
        ],
        compiler_params=pltpu.CompilerParams(use_tc_tiling_on_sc=False),
    )(_scatter_body)
    return f(tp, src, zsum, zcnt)


def _final_body(s_ref, c_ref, go_ref, na_ref, out_ref):
    s = s_ref[0] + s_ref[1]
    c = c_ref[0] + c_ref[1]
    cnt = jnp.clip(c[:, 0:1], 1.0, None)
    out_ref[...] = s / cnt * go_ref[...] + na_ref[...]


def _finalize(sums, cnts, go, node_attr):
    return pl.pallas_call(
        _final_body,
        out_shape=jax.ShapeDtypeStruct((N, OUT), jnp.float32),
    )(sums, cnts, go, node_attr)


def kernel(node_attr, edge_index, edge_attr, edge_sh, time, edge_time,
           W_fc_pre, b_fc_pre, W_ada, b_ada, W_fc1, b_fc1, W_fc2, b_fc2,
           W_tin, b_tin, W_tout, b_tout):
    Wcat = jnp.zeros((NEF + TD, 3 * NEF), jnp.float32)
    Wcat = Wcat.at[:NEF, :NEF].set(W_fc_pre).at[NEF:, NEF:].set(W_ada)
    bcat = jnp.concatenate([b_fc_pre, b_ada]).reshape(1, 3 * NEF)
    W2p = W_fc2.reshape(NEF, IN, OUT).transpose(1, 0, 2).reshape(IN, NEF * OUT)
    W2aug = jnp.concatenate([W2p, b_fc2.reshape(IN, OUT)], axis=1)

    src = edge_index[0]
    dst = edge_index[1]

    na, go = _node_gates(time, node_attr, W_tin, b_tin, W_tout, b_tout)
    xg = _gather_rows(na, dst)
    tp = _edge_tp(edge_attr, edge_time, edge_sh, xg, Wcat, bcat, W_fc1, b_fc1,
                  W2aug)
    sums, cnts = _scatter_mean_partials(tp, src)
    return _finalize(sums, cnts, go, node_attr)


# speedup vs baseline: 3.1763x; 1.5104x over previous
import functools
import math

import jax
import jax.numpy as jnp
from jax import lax
from jax.experimental import pallas as pl
from jax.experimental.pallas import tpu as pltpu
from jax.experimental.pallas import tpu_sc as plsc

N = 10000
E = 160000
IN = 32
OUT = 32
NEF = 16
TD = 128
INV_SQRT_FAN = 1.0 / math.sqrt(IN * 1)

NC = 2
NS = 16
NW = NC * NS
CHUNK = 128
NCHUNKS = E // CHUNK
ROWS_PER_TILE = N // NS

P = 8
BE = 3200
R = BE // P
GRID = E // BE


def _node_gates_body(t_ref, na_ref, wtin_ref, btin_ref, wtout_ref, btout_ref,
                     nag_ref, go_ref):
    t = t_ref[...]
    st = t * jax.nn.sigmoid(t)
    gi = jnp.dot(st, wtin_ref[...], preferred_element_type=jnp.float32) + btin_ref[...]
    go = jnp.dot(st, wtout_ref[...], preferred_element_type=jnp.float32) + btout_ref[...]
    nag_ref[...] = na_ref[...] * gi
    go_ref[...] = go


def _node_gates(time, node_attr, W_tin, b_tin, W_tout, b_tout):
    return pl.pallas_call(
        _node_gates_body,
        out_shape=(
            jax.ShapeDtypeStruct((N, IN), jnp.float32),
            jax.ShapeDtypeStruct((N, OUT), jnp.float32),
        ),
    )(time, node_attr, W_tin, b_tin.reshape(1, IN), W_tout, b_tout.reshape(1, OUT))


def _gather_body(na_hbm, idx_hbm, out_hbm, idx_v, rows_v, sem):
    wid = lax.axis_index("s") * NC + lax.axis_index("c")
    nch = NCHUNKS // NW + jnp.where(wid < NCHUNKS % NW, 1, 0)

    def body(c, carry):
        base = (wid + NW * c) * CHUNK
        pltpu.sync_copy(idx_hbm.at[pl.ds(base, CHUNK)], idx_v)
        pltpu.async_copy(na_hbm.at[idx_v], rows_v, sem).wait()
        pltpu.sync_copy(rows_v, out_hbm.at[pl.ds(base, CHUNK)])
        return carry

    lax.fori_loop(0, nch, body, 0)


def _gather_rows(na, dst):
    mesh = plsc.VectorSubcoreMesh(core_axis_name="c", subcore_axis_name="s")
    f = functools.partial(
        pl.kernel,
        out_type=jax.ShapeDtypeStruct((E, IN), jnp.float32),
        mesh=mesh,
        scratch_types=[
            pltpu.VMEM((CHUNK,), jnp.int32),
            pltpu.VMEM((CHUNK, IN), jnp.float32),
            pltpu.SemaphoreType.DMA,
        ],
        compiler_params=pltpu.CompilerParams(use_tc_tiling_on_sc=False),
    )(_gather_body)
    return f(na, dst)


def _edge_body(eap_ref, etp_ref, xp_ref, shx_ref, wada_ref, bada_ref,
               wpre_ref, bpre_ref, w1_ref, b1_ref, w2g_ref, hx_ref, cl_ref,
               xb_ref, tp_ref):
    etp = etp_ref[...]
    stp = etp * jax.nn.sigmoid(etp)
    adap = jnp.dot(stp, wada_ref[...], preferred_element_type=jnp.float32)
    adap = adap + bada_ref[...]
    prep = jnp.dot(eap_ref[...], wpre_ref[...],
                   preferred_element_type=jnp.float32) + bpre_ref[...]
    segs = []
    for k in range(P):
        shift = adap[:, 32 * k:32 * k + 16]
        scale = adap[:, 32 * k + 16:32 * k + 32]
        segs.append(prep[:, 16 * k:16 * k + 16] * (1.0 + scale) + shift)
    modp = jnp.concatenate(segs, axis=1)
    hp = jnp.dot(modp, w1_ref[...], preferred_element_type=jnp.float32)
    hp = jnp.maximum(hp + b1_ref[...], 0.0)
    xs = xp_ref[...] * shx_ref[...]
    gp = jnp.dot(xs, w2g_ref[...], preferred_element_type=jnp.float32)
    hb = jnp.dot(hp, hx_ref[...], preferred_element_type=jnp.float32)
    xbp = jnp.dot(xs, xb_ref[...], preferred_element_type=jnp.float32)
    tpp = jnp.dot(hb * gp, cl_ref[...], preferred_element_type=jnp.float32)
    tp_ref[...] = tpp + xbp


def _edge_tp(eap, etp, xp, shx, Wada_bd, bada_t, Wpre_bd, bpre_t, W1_bd, b1_t,
             W2G, HX, CL, XB):
    full = lambda s: pl.BlockSpec(s, lambda i: (0, 0))
    return pl.pallas_call(
        _edge_body,
        grid=(GRID,),
        in_specs=[
            pl.BlockSpec((R, P * NEF), lambda i: (i, 0)),
            pl.BlockSpec((R, P * TD), lambda i: (i, 0)),
            pl.BlockSpec((R, P * IN), lambda i: (i, 0)),
            pl.BlockSpec((R, P * IN), lambda i: (i, 0)),
            full((P * TD, P * 2 * NEF)),
            full((1, P * 2 * NEF)),
            full((P * NEF, P * NEF)),
            full((1, P * NEF)),
            full((P * NEF, P * NEF)),
            full((1, P * NEF)),
            full((P * IN, NEF * P * OUT)),
            full((P * NEF, NEF * P * OUT)),
            full((NEF * P * OUT, P * OUT)),
            full((P * IN, P * OUT)),
        ],
        out_specs=pl.BlockSpec((R, P * OUT), lambda i: (i, 0)),
        out_shape=jax.ShapeDtypeStruct((E // P, P * OUT), jnp.float32),
    )(eap, etp, xp, shx, Wada_bd, bada_t, Wpre_bd, bpre_t, W1_bd, b1_t,
      W2G, HX, CL, XB)


def _scatter_body(tp_hbm, src_hbm, zsum_hbm, zcnt_hbm, sums_hbm, cnts_hbm,
                  idx_v, tp_v, ones_v, acc_sh, cnt_sh):
    cid = lax.axis_index("c")
    sid = lax.axis_index("s")
    wid = sid * NC + cid
    row0 = sid * ROWS_PER_TILE

    def fill(i, carry):
        ones_v[i, :] = jnp.full((16,), 1.0, jnp.float32)
        return carry
    lax.fori_loop(0, CHUNK, fill, 0)

    pltpu.sync_copy(zsum_hbm.at[pl.ds(row0, ROWS_PER_TILE)],
                    acc_sh.at[pl.ds(row0, ROWS_PER_TILE)])
    pltpu.sync_copy(zcnt_hbm.at[pl.ds(row0, ROWS_PER_TILE)],
                    cnt_sh.at[pl.ds(row0, ROWS_PER_TILE)])
    plsc.subcore_barrier()

    nch = NCHUNKS // NW + jnp.where(wid < NCHUNKS % NW, 1, 0)

    def body(c, carry):
        base = (wid + NW * c) * CHUNK
        pltpu.sync_copy(src_hbm.at[pl.ds(base, CHUNK)], idx_v)
        pltpu.sync_copy(tp_hbm.at[pl.ds(base, CHUNK)], tp_v)
        pltpu.sync_copy(tp_v, acc_sh.at[idx_v], add=True)
        pltpu.sync_copy(ones_v, cnt_sh.at[idx_v], add=True)
        return carry

    lax.fori_loop(0, nch, body, 0)
    plsc.subcore_barrier()

    pltpu.sync_copy(acc_sh.at[pl.ds(row0, ROWS_PER_TILE)],
                    sums_hbm.at[cid, pl.ds(row0, ROWS_PER_TILE)])
    pltpu.sync_copy(cnt_sh.at[pl.ds(row0, ROWS_PER_TILE)],
                    cnts_hbm.at[cid, pl.ds(row0, ROWS_PER_TILE)])


def _scatter_mean_partials(tp, src):
    mesh = plsc.VectorSubcoreMesh(core_axis_name="c", subcore_axis_name="s")
    zsum = jnp.zeros((N, OUT), jnp.float32)
    zcnt = jnp.zeros((N, 16), jnp.float32)
    f = functools.partial(
        pl.kernel,
        out_type=(
            jax.ShapeDtypeStruct((NC, N, OUT), jnp.float32),
            jax.ShapeDtypeStruct((NC, N, 16), jnp.float32),
        ),
        mesh=mesh,
        scratch_types=[
            pltpu.VMEM((CHUNK,), jnp.int32),
            pltpu.VMEM((CHUNK, OUT), jnp.float32),
            pltpu.VMEM((CHUNK, 16), jnp.float32),
            pltpu.VMEM_SHARED((N, OUT), jnp.float32),
            pltpu.VMEM_SHARED((N, 16), jnp.float32),
        ],
        compiler_params=pltpu.CompilerParams(use_tc_tiling_on_sc=False),
    )(_scatter_body)
    return f(tp, src, zsum, zcnt)


def _final_body(s_ref, c_ref, go_ref, na_ref, out_ref):
    s = s_ref[0] + s_ref[1]
    c = c_ref[0] + c_ref[1]
    cnt = jnp.clip(c[:, 0:1], 1.0, None)
    out_ref[...] = s / cnt * go_ref[...] + na_ref[...]


def _finalize(sums, cnts, go, node_attr):
    return pl.pallas_call(
        _final_body,
        out_shape=jax.ShapeDtypeStruct((N, OUT), jnp.float32),
    )(sums, cnts, go, node_attr)


def kernel(node_attr, edge_index, edge_attr, edge_sh, time, edge_time,
           W_fc_pre, b_fc_pre, W_ada, b_ada, W_fc1, b_fc1, W_fc2, b_fc2,
           W_tin, b_tin, W_tout, b_tout):
    eyeP = jnp.eye(P, dtype=jnp.float32)
    Wada_bd = jnp.einsum('ab,tc->atbc', eyeP, W_ada).reshape(P * TD, P * 2 * NEF)
    bada_t = jnp.tile(b_ada, P).reshape(1, P * 2 * NEF)
    Wpre_bd = jnp.einsum('ab,fc->afbc', eyeP, W_fc_pre).reshape(P * NEF, P * NEF)
    bpre_t = jnp.tile(b_fc_pre, P).reshape(1, P * NEF)
    W1_bd = jnp.einsum('ab,fc->afbc', eyeP, W_fc1).reshape(P * NEF, P * NEF)
    b1_t = jnp.tile(b_fc1, P).reshape(1, P * NEF)
    W2t = W_fc2.reshape(NEF, IN, OUT).transpose(1, 0, 2)
    W2G = jnp.einsum('ab,ijm->aijbm', eyeP, W2t).reshape(P * IN, NEF * P * OUT)
    HX = jnp.einsum('ab,cd,m->acdbm', eyeP, jnp.eye(NEF, dtype=jnp.float32),
                    jnp.ones((OUT,), jnp.float32)).reshape(P * NEF, NEF * P * OUT)
    CL = jnp.einsum('j,ab,cd->jacbd', jnp.ones((NEF,), jnp.float32), eyeP,
                    jnp.eye(OUT, dtype=jnp.float32)).reshape(NEF * P * OUT, P * OUT)
    b2r = b_fc2.reshape(IN, OUT)
    XB = jnp.einsum('ab,im->aibm', eyeP, b2r).reshape(P * IN, P * OUT)

    eap = edge_attr.reshape(E // P, P * NEF)
    etp = edge_time.reshape(E // P, P * TD)
    shx = jnp.broadcast_to(edge_sh * INV_SQRT_FAN, (E, IN)).reshape(E // P, P * IN)

    src = edge_index[0]
    dst = edge_index[1]

    na, go = _node_gates(time, node_attr, W_tin, b_tin, W_tout, b_tout)
    xg = _gather_rows(na, dst)
    xp = xg.reshape(E // P, P * IN)
    tpp = _edge_tp(eap, etp, xp, shx, Wada_bd, bada_t, Wpre_bd, bpre_t,
                   W1_bd, b1_t, W2G, HX, CL, XB)
    tp = tpp.reshape(E, OUT)
    sums, cnts = _scatter_mean_partials(tp, src)
    return _finalize(sums, cnts, go, node_attr)


# speedup vs baseline: 3.6384x; 1.1455x over previous
import functools
import math

import jax
import jax.numpy as jnp
from jax import lax
from jax.experimental import pallas as pl
from jax.experimental.pallas import tpu as pltpu
from jax.experimental.pallas import tpu_sc as plsc

N = 10000
E = 160000
IN = 32
OUT = 32
NEF = 16
TD = 128
INV_SQRT_FAN = 1.0 / math.sqrt(IN * 1)

NC = 2
NS = 16
NW = NC * NS
CHUNK = 128
NCHUNKS = E // CHUNK
ROWS_PER_TILE = N // NS

P = 8
BE = 3200
R = BE // P
GRID = E // BE


def _node_gates_body(t_ref, na_ref, wtin_ref, btin_ref, wtout_ref, btout_ref,
                     nag_ref, go_ref):
    t = t_ref[...]
    st = t * jax.nn.sigmoid(t)
    gi = jnp.dot(st, wtin_ref[...], preferred_element_type=jnp.float32) + btin_ref[...]
    go = jnp.dot(st, wtout_ref[...], preferred_element_type=jnp.float32) + btout_ref[...]
    nag_ref[...] = na_ref[...] * gi
    go_ref[...] = go


def _node_gates(time, node_attr, W_tin, b_tin, W_tout, b_tout):
    return pl.pallas_call(
        _node_gates_body,
        out_shape=(
            jax.ShapeDtypeStruct((N, IN), jnp.float32),
            jax.ShapeDtypeStruct((N, OUT), jnp.float32),
        ),
    )(time, node_attr, W_tin, b_tin.reshape(1, IN), W_tout, b_tout.reshape(1, OUT))


def _gather_body(na_hbm, idx_hbm, out_hbm, idx_v, rows_v, sem):
    wid = lax.axis_index("s") * NC + lax.axis_index("c")
    nch = NCHUNKS // NW + jnp.where(wid < NCHUNKS % NW, 1, 0)

    def body(c, carry):
        base = (wid + NW * c) * CHUNK
        pltpu.sync_copy(idx_hbm.at[pl.ds(base, CHUNK)], idx_v)
        pltpu.async_copy(na_hbm.at[idx_v], rows_v, sem).wait()
        pltpu.sync_copy(rows_v, out_hbm.at[pl.ds(base, CHUNK)])
        return carry

    lax.fori_loop(0, nch, body, 0)


def _gather_rows(na, dst):
    mesh = plsc.VectorSubcoreMesh(core_axis_name="c", subcore_axis_name="s")
    f = functools.partial(
        pl.kernel,
        out_type=jax.ShapeDtypeStruct((E, IN), jnp.float32),
        mesh=mesh,
        scratch_types=[
            pltpu.VMEM((CHUNK,), jnp.int32),
            pltpu.VMEM((CHUNK, IN), jnp.float32),
            pltpu.SemaphoreType.DMA,
        ],
        compiler_params=pltpu.CompilerParams(use_tc_tiling_on_sc=False),
    )(_gather_body)
    return f(na, dst)


def _edge_body(eap_ref, etp_ref, xp_ref, shx_ref, wada_ref, bada_ref,
               wpre_ref, bpre_ref, w1_ref, b1_ref, w2g_ref, hx_ref,
               xb_ref, tp_ref):
    etp = etp_ref[...]
    stp = etp * jax.nn.sigmoid(etp)
    adap = jnp.dot(stp, wada_ref[...], preferred_element_type=jnp.float32)
    adap = adap + bada_ref[...]
    prep = jnp.dot(eap_ref[...], wpre_ref[...],
                   preferred_element_type=jnp.float32) + bpre_ref[...]
    segs = []
    for k in range(P):
        shift = adap[:, 32 * k:32 * k + 16]
        scale = adap[:, 32 * k + 16:32 * k + 32]
        segs.append(prep[:, 16 * k:16 * k + 16] * (1.0 + scale) + shift)
    modp = jnp.concatenate(segs, axis=1)
    hp = jnp.dot(modp, w1_ref[...], preferred_element_type=jnp.float32)
    hp = jnp.maximum(hp + b1_ref[...], 0.0)
    xs = xp_ref[...] * shx_ref[...]
    acc = jnp.dot(xs, xb_ref[...], preferred_element_type=jnp.float32)
    for j in range(NEF):
        gj = jnp.dot(xs, w2g_ref[:, j * P * OUT:(j + 1) * P * OUT],
                     preferred_element_type=jnp.float32)
        hj = jnp.dot(hp, hx_ref[:, j * P * OUT:(j + 1) * P * OUT],
                     preferred_element_type=jnp.float32)
        acc = acc + gj * hj
    tp_ref[...] = acc


def _edge_tp(eap, etp, xp, shx, Wada_bd, bada_t, Wpre_bd, bpre_t, W1_bd, b1_t,
             W2G, HX, XB):
    full = lambda s: pl.BlockSpec(s, lambda i: (0, 0))
    return pl.pallas_call(
        _edge_body,
        grid=(GRID,),
        in_specs=[
            pl.BlockSpec((R, P * NEF), lambda i: (i, 0)),
            pl.BlockSpec((R, P * TD), lambda i: (i, 0)),
            pl.BlockSpec((R, P * IN), lambda i: (i, 0)),
            pl.BlockSpec((R, P * IN), lambda i: (i, 0)),
            full((P * TD, P * 2 * NEF)),
            full((1, P * 2 * NEF)),
            full((P * NEF, P * NEF)),
            full((1, P * NEF)),
            full((P * NEF, P * NEF)),
            full((1, P * NEF)),
            full((P * IN, NEF * P * OUT)),
            full((P * NEF, NEF * P * OUT)),
            full((P * IN, P * OUT)),
        ],
        out_specs=pl.BlockSpec((R, P * OUT), lambda i: (i, 0)),
        out_shape=jax.ShapeDtypeStruct((E // P, P * OUT), jnp.float32),
    )(eap, etp, xp, shx, Wada_bd, bada_t, Wpre_bd, bpre_t, W1_bd, b1_t,
      W2G, HX, XB)


def _scatter_body(tp_hbm, src_hbm, zsum_hbm, zcnt_hbm, sums_hbm, cnts_hbm,
                  idx_v, tp_v, ones_v, acc_sh, cnt_sh):
    cid = lax.axis_index("c")
    sid = lax.axis_index("s")
    wid = sid * NC + cid
    row0 = sid * ROWS_PER_TILE

    def fill(i, carry):
        ones_v[i, :] = jnp.full((16,), 1.0, jnp.float32)
        return carry
    lax.fori_loop(0, CHUNK, fill, 0)

    pltpu.sync_copy(zsum_hbm.at[pl.ds(row0, ROWS_PER_TILE)],
                    acc_sh.at[pl.ds(row0, ROWS_PER_TILE)])
    pltpu.sync_copy(zcnt_hbm.at[pl.ds(row0, ROWS_PER_TILE)],
                    cnt_sh.at[pl.ds(row0, ROWS_PER_TILE)])
    plsc.subcore_barrier()

    nch = NCHUNKS // NW + jnp.where(wid < NCHUNKS % NW, 1, 0)

    def body(c, carry):
        base = (wid + NW * c) * CHUNK
        pltpu.sync_copy(src_hbm.at[pl.ds(base, CHUNK)], idx_v)
        pltpu.sync_copy(tp_hbm.at[pl.ds(base, CHUNK)], tp_v)
        pltpu.sync_copy(tp_v, acc_sh.at[idx_v], add=True)
        pltpu.sync_copy(ones_v, cnt_sh.at[idx_v], add=True)
        return carry

    lax.fori_loop(0, nch, body, 0)
    plsc.subcore_barrier()

    pltpu.sync_copy(acc_sh.at[pl.ds(row0, ROWS_PER_TILE)],
                    sums_hbm.at[cid, pl.ds(row0, ROWS_PER_TILE)])
    pltpu.sync_copy(cnt_sh.at[pl.ds(row0, ROWS_PER_TILE)],
                    cnts_hbm.at[cid, pl.ds(row0, ROWS_PER_TILE)])


def _scatter_mean_partials(tp, src):
    mesh = plsc.VectorSubcoreMesh(core_axis_name="c", subcore_axis_name="s")
    zsum = jnp.zeros((N, OUT), jnp.float32)
    zcnt = jnp.zeros((N, 16), jnp.float32)
    f = functools.partial(
        pl.kernel,
        out_type=(
            jax.ShapeDtypeStruct((NC, N, OUT), jnp.float32),
            jax.ShapeDtypeStruct((NC, N, 16), jnp.float32),
        ),
        mesh=mesh,
        scratch_types=[
            pltpu.VMEM((CHUNK,), jnp.int32),
            pltpu.VMEM((CHUNK, OUT), jnp.float32),
            pltpu.VMEM((CHUNK, 16), jnp.float32),
            pltpu.VMEM_SHARED((N, OUT), jnp.float32),
            pltpu.VMEM_SHARED((N, 16), jnp.float32),
        ],
        compiler_params=pltpu.CompilerParams(use_tc_tiling_on_sc=False),
    )(_scatter_body)
    return f(tp, src, zsum, zcnt)


def _final_body(s_ref, c_ref, go_ref, na_ref, out_ref):
    s = s_ref[0] + s_ref[1]
    c = c_ref[0] + c_ref[1]
    cnt = jnp.clip(c[:, 0:1], 1.0, None)
    out_ref[...] = s / cnt * go_ref[...] + na_ref[...]


def _finalize(sums, cnts, go, node_attr):
    return pl.pallas_call(
        _final_body,
        out_shape=jax.ShapeDtypeStruct((N, OUT), jnp.float32),
    )(sums, cnts, go, node_attr)


def kernel(node_attr, edge_index, edge_attr, edge_sh, time, edge_time,
           W_fc_pre, b_fc_pre, W_ada, b_ada, W_fc1, b_fc1, W_fc2, b_fc2,
           W_tin, b_tin, W_tout, b_tout):
    eyeP = jnp.eye(P, dtype=jnp.float32)
    Wada_bd = jnp.einsum('ab,tc->atbc', eyeP, W_ada).reshape(P * TD, P * 2 * NEF)
    bada_t = jnp.tile(b_ada, P).reshape(1, P * 2 * NEF)
    Wpre_bd = jnp.einsum('ab,fc->afbc', eyeP, W_fc_pre).reshape(P * NEF, P * NEF)
    bpre_t = jnp.tile(b_fc_pre, P).reshape(1, P * NEF)
    W1_bd = jnp.einsum('ab,fc->afbc', eyeP, W_fc1).reshape(P * NEF, P * NEF)
    b1_t = jnp.tile(b_fc1, P).reshape(1, P * NEF)
    W2t = W_fc2.reshape(NEF, IN, OUT).transpose(1, 0, 2)
    W2G = jnp.einsum('ab,ijm->aijbm', eyeP, W2t).reshape(P * IN, NEF * P * OUT)
    HX = jnp.einsum('ab,cd,m->acdbm', eyeP, jnp.eye(NEF, dtype=jnp.float32),
                    jnp.ones((OUT,), jnp.float32)).reshape(P * NEF, NEF * P * OUT)
    b2r = b_fc2.reshape(IN, OUT)
    XB = jnp.einsum('ab,im->aibm', eyeP, b2r).reshape(P * IN, P * OUT)

    eap = edge_attr.reshape(E // P, P * NEF)
    etp = edge_time.reshape(E // P, P * TD)
    shx = jnp.broadcast_to(edge_sh * INV_SQRT_FAN, (E, IN)).reshape(E // P, P * IN)

    src = edge_index[0]
    dst = edge_index[1]

    na, go = _node_gates(time, node_attr, W_tin, b_tin, W_tout, b_tout)
    xg = _gather_rows(na, dst)
    xp = xg.reshape(E // P, P * IN)
    tpp = _edge_tp(eap, etp, xp, shx, Wada_bd, bada_t, Wpre_bd, bpre_t,
                   W1_bd, b1_t, W2G, HX, XB)
    tp = tpp.reshape(E, OUT)
    sums, cnts = _scatter_mean_partials(tp, src)
    return _finalize(sums, cnts, go, node_attr)


# speedup vs baseline: 3.7002x; 1.0170x over previous
import functools
import math

import jax
import jax.numpy as jnp
from jax import lax
from jax.experimental import pallas as pl
from jax.experimental.pallas import tpu as pltpu
from jax.experimental.pallas import tpu_sc as plsc

N = 10000
E = 160000
IN = 32
OUT = 32
NEF = 16
TD = 128
INV_SQRT_FAN = 1.0 / math.sqrt(IN * 1)

NC = 2
NS = 16
NW = NC * NS
CHUNK = 128
NCHUNKS = E // CHUNK
ROWS_PER_TILE = N // NS

P = 8
BE = 6400
R = BE // P
GRID = E // BE


def _node_gates_body(t_ref, na_ref, wtin_ref, btin_ref, wtout_ref, btout_ref,
                     nag_ref, go_ref):
    t = t_ref[...]
    st = t * jax.nn.sigmoid(t)
    gi = jnp.dot(st, wtin_ref[...], preferred_element_type=jnp.float32) + btin_ref[...]
    go = jnp.dot(st, wtout_ref[...], preferred_element_type=jnp.float32) + btout_ref[...]
    nag_ref[...] = na_ref[...] * gi
    go_ref[...] = go


def _node_gates(time, node_attr, W_tin, b_tin, W_tout, b_tout):
    return pl.pallas_call(
        _node_gates_body,
        out_shape=(
            jax.ShapeDtypeStruct((N, IN), jnp.float32),
            jax.ShapeDtypeStruct((N, OUT), jnp.float32),
        ),
    )(time, node_attr, W_tin, b_tin.reshape(1, IN), W_tout, b_tout.reshape(1, OUT))


def _gather_body(na_hbm, idx_hbm, out_hbm, idx_v, rows_v, sem):
    wid = lax.axis_index("s") * NC + lax.axis_index("c")
    nch = NCHUNKS // NW + jnp.where(wid < NCHUNKS % NW, 1, 0)

    def body(c, carry):
        base = (wid + NW * c) * CHUNK
        pltpu.sync_copy(idx_hbm.at[pl.ds(base, CHUNK)], idx_v)
        pltpu.async_copy(na_hbm.at[idx_v], rows_v, sem).wait()
        pltpu.sync_copy(rows_v, out_hbm.at[pl.ds(base, CHUNK)])
        return carry

    lax.fori_loop(0, nch, body, 0)


def _gather_rows(na, dst):
    mesh = plsc.VectorSubcoreMesh(core_axis_name="c", subcore_axis_name="s")
    f = functools.partial(
        pl.kernel,
        out_type=jax.ShapeDtypeStruct((E, IN), jnp.float32),
        mesh=mesh,
        scratch_types=[
            pltpu.VMEM((CHUNK,), jnp.int32),
            pltpu.VMEM((CHUNK, IN), jnp.float32),
            pltpu.SemaphoreType.DMA,
        ],
        compiler_params=pltpu.CompilerParams(use_tc_tiling_on_sc=False),
    )(_gather_body)
    return f(na, dst)


def _edge_body(eap_ref, etp_ref, xp_ref, shx_ref, wada_ref, bada_ref,
               wpre_ref, bpre_ref, w1_ref, b1_ref, w2g_ref, hx_ref,
               xb_ref, tp_ref):
    etp = etp_ref[...]
    stp = etp * jax.nn.sigmoid(etp)
    adap = jnp.dot(stp, wada_ref[...], preferred_element_type=jnp.float32)
    adap = adap + bada_ref[...]
    prep = jnp.dot(eap_ref[...], wpre_ref[...],
                   preferred_element_type=jnp.float32) + bpre_ref[...]
    segs = []
    for k in range(P):
        shift = adap[:, 32 * k:32 * k + 16]
        scale = adap[:, 32 * k + 16:32 * k + 32]
        segs.append(prep[:, 16 * k:16 * k + 16] * (1.0 + scale) + shift)
    modp = jnp.concatenate(segs, axis=1)
    hp = jnp.dot(modp, w1_ref[...], preferred_element_type=jnp.float32)
    hp = jnp.maximum(hp + b1_ref[...], 0.0)
    xs = xp_ref[...] * shx_ref[...]
    acc = jnp.dot(xs, xb_ref[...], preferred_element_type=jnp.float32)
    for j in range(NEF):
        gj = jnp.dot(xs, w2g_ref[:, j * P * OUT:(j + 1) * P * OUT],
                     preferred_element_type=jnp.float32)
        hj = jnp.dot(hp, hx_ref[:, j * P * OUT:(j + 1) * P * OUT],
                     preferred_element_type=jnp.float32)
        acc = acc + gj * hj
    tp_ref[...] = acc


def _edge_tp(eap, etp, xp, shx, Wada_bd, bada_t, Wpre_bd, bpre_t, W1_bd, b1_t,
             W2G, HX, XB):
    full = lambda s: pl.BlockSpec(s, lambda i: (0, 0))
    return pl.pallas_call(
        _edge_body,
        grid=(GRID,),
        in_specs=[
            pl.BlockSpec((R, P * NEF), lambda i: (i, 0)),
            pl.BlockSpec((R, P * TD), lambda i: (i, 0)),
            pl.BlockSpec((R, P * IN), lambda i: (i, 0)),
            pl.BlockSpec((R, P * IN), lambda i: (i, 0)),
            full((P * TD, P * 2 * NEF)),
            full((1, P * 2 * NEF)),
            full((P * NEF, P * NEF)),
            full((1, P * NEF)),
            full((P * NEF, P * NEF)),
            full((1, P * NEF)),
            full((P * IN, NEF * P * OUT)),
            full((P * NEF, NEF * P * OUT)),
            full((P * IN, P * OUT)),
        ],
        out_specs=pl.BlockSpec((R, P * OUT), lambda i: (i, 0)),
        out_shape=jax.ShapeDtypeStruct((E // P, P * OUT), jnp.float32),
    )(eap, etp, xp, shx, Wada_bd, bada_t, Wpre_bd, bpre_t, W1_bd, b1_t,
      W2G, HX, XB)


def _scatter_body(tp_hbm, src_hbm, zsum_hbm, zcnt_hbm, sums_hbm, cnts_hbm,
                  idx_v, tp_v, ones_v, acc_sh, cnt_sh):
    cid = lax.axis_index("c")
    sid = lax.axis_index("s")
    wid = sid * NC + cid
    row0 = sid * ROWS_PER_TILE

    def fill(i, carry):
        ones_v[i, :] = jnp.full((16,), 1.0, jnp.float32)
        return carry
    lax.fori_loop(0, CHUNK, fill, 0)

    pltpu.sync_copy(zsum_hbm.at[pl.ds(row0, ROWS_PER_TILE)],
                    acc_sh.at[pl.ds(row0, ROWS_PER_TILE)])
    pltpu.sync_copy(zcnt_hbm.at[pl.ds(row0, ROWS_PER_TILE)],
                    cnt_sh.at[pl.ds(row0, ROWS_PER_TILE)])
    plsc.subcore_barrier()

    nch = NCHUNKS // NW + jnp.where(wid < NCHUNKS % NW, 1, 0)

    def body(c, carry):
        base = (wid + NW * c) * CHUNK
        pltpu.sync_copy(src_hbm.at[pl.ds(base, CHUNK)], idx_v)
        pltpu.sync_copy(tp_hbm.at[pl.ds(base, CHUNK)], tp_v)
        pltpu.sync_copy(tp_v, acc_sh.at[idx_v], add=True)
        pltpu.sync_copy(ones_v, cnt_sh.at[idx_v], add=True)
        return carry

    lax.fori_loop(0, nch, body, 0)
    plsc.subcore_barrier()

    pltpu.sync_copy(acc_sh.at[pl.ds(row0, ROWS_PER_TILE)],
                    sums_hbm.at[cid, pl.ds(row0, ROWS_PER_TILE)])
    pltpu.sync_copy(cnt_sh.at[pl.ds(row0, ROWS_PER_TILE)],
                    cnts_hbm.at[cid, pl.ds(row0, ROWS_PER_TILE)])


def _scatter_mean_partials(tp, src):
    mesh = plsc.VectorSubcoreMesh(core_axis_name="c", subcore_axis_name="s")
    zsum = jnp.zeros((N, OUT), jnp.float32)
    zcnt = jnp.zeros((N, 16), jnp.float32)
    f = functools.partial(
        pl.kernel,
        out_type=(
            jax.ShapeDtypeStruct((NC, N, OUT), jnp.float32),
            jax.ShapeDtypeStruct((NC, N, 16), jnp.float32),
        ),
        mesh=mesh,
        scratch_types=[
            pltpu.VMEM((CHUNK,), jnp.int32),
            pltpu.VMEM((CHUNK, OUT), jnp.float32),
            pltpu.VMEM((CHUNK, 16), jnp.float32),
            pltpu.VMEM_SHARED((N, OUT), jnp.float32),
            pltpu.VMEM_SHARED((N, 16), jnp.float32),
        ],
        compiler_params=pltpu.CompilerParams(use_tc_tiling_on_sc=False),
    )(_scatter_body)
    return f(tp, src, zsum, zcnt)


def _final_body(s_ref, c_ref, go_ref, na_ref, out_ref):
    s = s_ref[0] + s_ref[1]
    c = c_ref[0] + c_ref[1]
    cnt = jnp.clip(c[:, 0:1], 1.0, None)
    out_ref[...] = s / cnt * go_ref[...] + na_ref[...]


def _finalize(sums, cnts, go, node_attr):
    return pl.pallas_call(
        _final_body,
        out_shape=jax.ShapeDtypeStruct((N, OUT), jnp.float32),
    )(sums, cnts, go, node_attr)


def kernel(node_attr, edge_index, edge_attr, edge_sh, time, edge_time,
           W_fc_pre, b_fc_pre, W_ada, b_ada, W_fc1, b_fc1, W_fc2, b_fc2,
           W_tin, b_tin, W_tout, b_tout):
    eyeP = jnp.eye(P, dtype=jnp.float32)
    Wada_bd = jnp.einsum('ab,tc->atbc', eyeP, W_ada).reshape(P * TD, P * 2 * NEF)
    bada_t = jnp.tile(b_ada, P).reshape(1, P * 2 * NEF)
    Wpre_bd = jnp.einsum('ab,fc->afbc', eyeP, W_fc_pre).reshape(P * NEF, P * NEF)
    bpre_t = jnp.tile(b_fc_pre, P).reshape(1, P * NEF)
    W1_bd = jnp.einsum('ab,fc->afbc', eyeP, W_fc1).reshape(P * NEF, P * NEF)
    b1_t = jnp.tile(b_fc1, P).reshape(1, P * NEF)
    W2t = W_fc2.reshape(NEF, IN, OUT).transpose(1, 0, 2)
    W2G = jnp.einsum('ab,ijm->aijbm', eyeP, W2t).reshape(P * IN, NEF * P * OUT)
    HX = jnp.einsum('ab,cd,m->acdbm', eyeP, jnp.eye(NEF, dtype=jnp.float32),
                    jnp.ones((OUT,), jnp.float32)).reshape(P * NEF, NEF * P * OUT)
    b2r = b_fc2.reshape(IN, OUT)
    XB = jnp.einsum('ab,im->aibm', eyeP, b2r).reshape(P * IN, P * OUT)

    eap = edge_attr.reshape(E // P, P * NEF)
    etp = edge_time.reshape(E // P, P * TD)
    shx = jnp.broadcast_to(edge_sh * INV_SQRT_FAN, (E, IN)).reshape(E // P, P * IN)

    src = edge_index[0]
    dst = edge_index[1]

    na, go = _node_gates(time, node_attr, W_tin, b_tin, W_tout, b_tout)
    xg = _gather_rows(na, dst)
    xp = xg.reshape(E // P, P * IN)
    tpp = _edge_tp(eap, etp, xp, shx, Wada_bd, bada_t, Wpre_bd, bpre_t,
                   W1_bd, b1_t, W2G, HX, XB)
    tp = tpp.reshape(E, OUT)
    sums, cnts = _scatter_mean_partials(tp, src)
    return _finalize(sums, cnts, go, node_attr)
